# baseline (speedup 1.0000x reference)
import functools

import jax
import jax.numpy as jnp
from jax import lax
from jax.experimental import pallas as pl
from jax.experimental.pallas import tpu as pltpu
from jax.experimental.pallas import tpu_sc as plsc

DEPTH = 4
N_ATOMS = 10000
N_BONDS = 320000
MAX_NB = 32
ATOM_FDIM = 128
BOND_FDIM = 144
HIDDEN = 64

NC, NS = 2, 16
NW = NC * NS
CH = 128
NB_PAD = 327680
NA_PAD = 10240
BOND_BLK = 4096
ATOM_BLK = 512


def _sc_gather(table, idx2d):
    n_idx_rows, ch = idx2d.shape
    B = n_idx_rows * ch
    D = table.shape[1]
    b_per_w = B // NW
    n_ch = b_per_w // CH
    assert b_per_w % CH == 0 and n_ch % 2 == 0
    mesh = plsc.VectorSubcoreMesh(core_axis_name="c", subcore_axis_name="s")

    @functools.partial(
        pl.kernel,
        out_type=jax.ShapeDtypeStruct((B, D), jnp.float32),
        mesh=mesh,
        compiler_params=pltpu.CompilerParams(use_tc_tiling_on_sc=False),
        scratch_types=[
            pltpu.VMEM((n_ch, CH), jnp.int32),
            pltpu.VMEM((CH, D), jnp.float32),
            pltpu.VMEM((CH, D), jnp.float32),
            pltpu.SemaphoreType.DMA,
            pltpu.SemaphoreType.DMA,
        ],
    )
    def gather_k(table_hbm, idx_hbm, out_hbm, idx_v, buf0, buf1, sem0, sem1):
        wid = lax.axis_index("s") * NC + lax.axis_index("c")
        base = wid * b_per_w
        pltpu.sync_copy(idx_hbm.at[pl.ds(wid * n_ch, n_ch)], idx_v)

        def _start(i, buf, sem):
            pltpu.async_copy(table_hbm.at[idx_v.at[i]], buf, sem)

        def _wait(buf, sem):
            pltpu.make_async_copy(table_hbm.at[pl.ds(0, CH)], buf, sem).wait()

        def _put(i, buf):
            pltpu.sync_copy(buf, out_hbm.at[pl.ds(base + i * CH, CH)])

        _start(0, buf0, sem0)

        def outer(g, carry):
            i0 = g * 2
            _start(i0 + 1, buf1, sem1)
            _wait(buf0, sem0)
            _put(i0, buf0)

            @pl.when(i0 + 2 < n_ch)
            def _():
                _start(i0 + 2, buf0, sem0)

            _wait(buf1, sem1)
            _put(i0 + 1, buf1)
            return carry

        lax.fori_loop(0, n_ch // 2, outer, 0)

    return gather_k(table, idx2d)


def _mm_in(f_bonds, w1b_t, bg1, wg2_t, bg2):
    blk = 2560
    grid = N_BONDS // blk

    def body(fb_ref, w_ref, b1_ref, w2_ref, b2_ref, fbp_ref, msg_ref):
        fbp = jnp.dot(fb_ref[...], w_ref[...], preferred_element_type=jnp.float32)
        fbp = fbp + b1_ref[...]
        fbp_ref[...] = fbp
        h = jnp.maximum(fbp, 0.0)
        m = jnp.dot(h, w2_ref[...], preferred_element_type=jnp.float32) + b2_ref[...]
        rows = lax.broadcasted_iota(jnp.int32, m.shape, 0)
        m = jnp.where(jnp.logical_and(rows == 0, pl.program_id(0) == 0), 0.0, m)
        msg_ref[...] = m

    return pl.pallas_call(
        body,
        grid=(grid,),
        in_specs=[
            pl.BlockSpec((blk, BOND_FDIM), lambda i: (i, 0)),
            pl.BlockSpec((BOND_FDIM, HIDDEN), lambda i: (0, 0)),
            pl.BlockSpec((1, HIDDEN), lambda i: (0, 0)),
            pl.BlockSpec((HIDDEN, HIDDEN), lambda i: (0, 0)),
            pl.BlockSpec((1, HIDDEN), lambda i: (0, 0)),
        ],
        out_specs=[
            pl.BlockSpec((blk, HIDDEN), lambda i: (i, 0)),
            pl.BlockSpec((blk, HIDDEN), lambda i: (i, 0)),
        ],
        out_shape=[
            jax.ShapeDtypeStruct((NB_PAD, HIDDEN), jnp.float32),
            jax.ShapeDtypeStruct((NB_PAD, HIDDEN), jnp.float32),
        ],
    )(f_bonds, w1b_t, bg1, wg2_t, bg2)


def _nei_sum(g3d, plane):
    grid = NA_PAD // ATOM_BLK

    def body(g_ref, out_ref):
        out_ref[...] = jnp.sum(g_ref[...], axis=0)

    return pl.pallas_call(
        body,
        grid=(grid,),
        in_specs=[pl.BlockSpec((MAX_NB, ATOM_BLK, HIDDEN), lambda i: (plane, i, 0))],
        out_specs=pl.BlockSpec((ATOM_BLK, HIDDEN), lambda i: (i, 0)),
        out_shape=jax.ShapeDtypeStruct((NA_PAD, HIDDEN), jnp.float32),
    )(g3d)


def _depth_update(fbp, ag, gcat, wmh_t, wg2_t, bg2):
    grid = NB_PAD // BOND_BLK

    def body(fbp_ref, ag_ref, rev_ref, wm_ref, w2_ref, b2_ref, out_ref):
        delta = ag_ref[...] - rev_ref[...]
        h = fbp_ref[...] + jnp.dot(delta, wm_ref[...], preferred_element_type=jnp.float32)
        h = jnp.maximum(h, 0.0)
        m = jnp.dot(h, w2_ref[...], preferred_element_type=jnp.float32) + b2_ref[...]
        rows = lax.broadcasted_iota(jnp.int32, m.shape, 0)
        m = jnp.where(jnp.logical_and(rows == 0, pl.program_id(0) == 0), 0.0, m)
        out_ref[...] = m

    return pl.pallas_call(
        body,
        grid=(grid,),
        in_specs=[
            pl.BlockSpec((BOND_BLK, HIDDEN), lambda i: (i, 0)),
            pl.BlockSpec((BOND_BLK, HIDDEN), lambda i: (i, 0)),
            pl.BlockSpec((BOND_BLK, HIDDEN), lambda i: (i, 0)),
            pl.BlockSpec((HIDDEN, HIDDEN), lambda i: (0, 0)),
            pl.BlockSpec((HIDDEN, HIDDEN), lambda i: (0, 0)),
            pl.BlockSpec((1, HIDDEN), lambda i: (0, 0)),
        ],
        out_specs=pl.BlockSpec((BOND_BLK, HIDDEN), lambda i: (i, 0)),
        out_shape=jax.ShapeDtypeStruct((NB_PAD, HIDDEN), jnp.float32),
    )(fbp, ag, gcat, wmh_t, wg2_t, bg2)


def _final_mlp(msgs, wm1_t, bm1, wm2_t, bm2):
    grid = NB_PAD // BOND_BLK
    H2 = 2 * HIDDEN

    def body(m0, m1, m2, m3, w1_ref, b1_ref, w2_ref, b2_ref, out_ref):
        s = jnp.dot(m0[...], w1_ref[0 * HIDDEN:1 * HIDDEN, :], preferred_element_type=jnp.float32)
        s += jnp.dot(m1[...], w1_ref[1 * HIDDEN:2 * HIDDEN, :], preferred_element_type=jnp.float32)
        s += jnp.dot(m2[...], w1_ref[2 * HIDDEN:3 * HIDDEN, :], preferred_element_type=jnp.float32)
        s += jnp.dot(m3[...], w1_ref[3 * HIDDEN:4 * HIDDEN, :], preferred_element_type=jnp.float32)
        h = jnp.maximum(s + b1_ref[...], 0.0)
        out_ref[...] = jnp.dot(h, w2_ref[...], preferred_element_type=jnp.float32) + b2_ref[...]

    mspec = pl.BlockSpec((BOND_BLK, HIDDEN), lambda i: (i, 0))
    return pl.pallas_call(
        body,
        grid=(grid,),
        in_specs=[
            mspec, mspec, mspec, mspec,
            pl.BlockSpec((DEPTH * HIDDEN, H2), lambda i: (0, 0)),
            pl.BlockSpec((1, H2), lambda i: (0, 0)),
            pl.BlockSpec((H2, HIDDEN), lambda i: (0, 0)),
            pl.BlockSpec((1, HIDDEN), lambda i: (0, 0)),
        ],
        out_specs=pl.BlockSpec((BOND_BLK, HIDDEN), lambda i: (i, 0)),
        out_shape=jax.ShapeDtypeStruct((NB_PAD, HIDDEN), jnp.float32),
    )(*msgs, wm1_t, bm1, wm2_t, bm2)


def _out_layer(gfin3d, fa_pad, woa_t, wom_t, bo):
    grid = NA_PAD // ATOM_BLK

    def body(g_ref, fa_ref, wa_ref, wm_ref, b_ref, out_ref):
        asum = jnp.sum(g_ref[...], axis=0)
        x = jnp.dot(fa_ref[...], wa_ref[...], preferred_element_type=jnp.float32)
        x += jnp.dot(asum, wm_ref[...], preferred_element_type=jnp.float32)
        out_ref[...] = jnp.maximum(x + b_ref[...], 0.0)

    return pl.pallas_call(
        body,
        grid=(grid,),
        in_specs=[
            pl.BlockSpec((MAX_NB, ATOM_BLK, HIDDEN), lambda i: (0, i, 0)),
            pl.BlockSpec((ATOM_BLK, ATOM_FDIM), lambda i: (i, 0)),
            pl.BlockSpec((ATOM_FDIM, HIDDEN), lambda i: (0, 0)),
            pl.BlockSpec((HIDDEN, HIDDEN), lambda i: (0, 0)),
            pl.BlockSpec((1, HIDDEN), lambda i: (0, 0)),
        ],
        out_specs=pl.BlockSpec((ATOM_BLK, HIDDEN), lambda i: (i, 0)),
        out_shape=jax.ShapeDtypeStruct((NA_PAD, HIDDEN), jnp.float32),
    )(gfin3d, fa_pad, woa_t, wom_t, bo)


def kernel(f_atoms, f_bonds, a2b, b2a, b2revb, undirected_b2a,
           W_g1, b_g1, W_g2, b_g2, W_m1, b_m1, W_m2, b_m2, W_o, b_o):
    del undirected_b2a
    w1b_t = W_g1[:, :BOND_FDIM].T
    wmh_t = W_g1[:, BOND_FDIM:].T
    wg2_t = W_g2.T
    wm1_t = W_m1.T
    wm2_t = W_m2.T
    woa_t = W_o[:, :ATOM_FDIM].T
    wom_t = W_o[:, ATOM_FDIM:].T
    bg1 = b_g1[None, :]
    bg2 = b_g2[None, :]
    bm1 = b_m1[None, :]
    bm2 = b_m2[None, :]
    bo = b_o[None, :]

    a2b_p = jnp.pad(a2b, ((0, NA_PAD - N_ATOMS), (0, 0)))
    idx_nei = a2b_p.T.reshape(-1)
    b2revb_p = jnp.pad(b2revb, (0, NB_PAD - N_BONDS))
    b2a_p = jnp.pad(b2a, (0, NB_PAD - N_BONDS))
    idx_cat = jnp.concatenate([b2revb_p, idx_nei])
    idx_cat2d = idx_cat.reshape(-1, CH)
    idx_nei2d = idx_nei.reshape(-1, CH)
    b2a2d = b2a_p.reshape(-1, CH)
    fa_pad = jnp.pad(f_atoms, ((0, NA_PAD - N_ATOMS), (0, 0)))

    fbp, msg = _mm_in(f_bonds, w1b_t, bg1, wg2_t, bg2)
    msgs = [msg]
    for _ in range(DEPTH - 1):
        gcat = _sc_gather(msg, idx_cat2d)
        g3d = gcat.reshape(2 * MAX_NB, NA_PAD, HIDDEN)
        a_msg = _nei_sum(g3d, 1)
        ag = _sc_gather(a_msg, b2a2d)
        msg = _depth_update(fbp, ag, gcat, wmh_t, wg2_t, bg2)
        msgs.append(msg)

    tmp = _final_mlp(msgs, wm1_t, bm1, wm2_t, bm2)
    gfin = _sc_gather(tmp, idx_nei2d)
    gfin3d = gfin.reshape(MAX_NB, NA_PAD, HIDDEN)
    out_pad = _out_layer(gfin3d, fa_pad, woa_t, wom_t, bo)
    return out_pad[:N_ATOMS]


# speedup vs baseline: 1.5487x; 1.5487x over previous
import functools

import jax
import jax.numpy as jnp
from jax import lax
from jax.experimental import pallas as pl
from jax.experimental.pallas import tpu as pltpu
from jax.experimental.pallas import tpu_sc as plsc

DEPTH = 4
N_ATOMS = 10000
N_BONDS = 320000
MAX_NB = 32
ATOM_FDIM = 128
BOND_FDIM = 144
HIDDEN = 64

NC, NS = 2, 16
NW = NC * NS
CH = 128
NB_PAD = 327680
NA_PAD = 10240
BOND_BLK = 4096
ATOM_BLK = 512


_A_PER_CH = CH // MAX_NB
_HV = HIDDEN // 16


def _sc_gather_sum(table, idx2d):
    D = table.shape[1]
    rows_per_w = NB_PAD // NW
    n_ch = rows_per_w // CH
    a_per_w = NA_PAD // NW
    mesh = plsc.VectorSubcoreMesh(core_axis_name="c", subcore_axis_name="s")

    @functools.partial(
        pl.kernel,
        out_type=jax.ShapeDtypeStruct((NA_PAD, D), jnp.float32),
        mesh=mesh,
        compiler_params=pltpu.CompilerParams(use_tc_tiling_on_sc=False),
        scratch_types=[
            pltpu.VMEM((n_ch, CH), jnp.int32),
            pltpu.VMEM((CH, D), jnp.float32),
            pltpu.VMEM((CH, D), jnp.float32),
            pltpu.VMEM((a_per_w, D), jnp.float32),
            pltpu.SemaphoreType.DMA,
            pltpu.SemaphoreType.DMA,
        ],
    )
    def gsum_k(table_hbm, idx_hbm, out_hbm, idx_v, buf0, buf1, acc_v, sem0, sem1):
        wid = lax.axis_index("s") * NC + lax.axis_index("c")
        pltpu.sync_copy(idx_hbm.at[pl.ds(wid * n_ch, n_ch)], idx_v)

        def _start(i, buf, sem):
            pltpu.async_copy(table_hbm.at[idx_v.at[i]], buf, sem)

        def _wait(buf, sem):
            pltpu.make_async_copy(table_hbm.at[pl.ds(0, CH)], buf, sem).wait()

        def _reduce(i, buf):
            for a in range(_A_PER_CH):
                r0 = a * MAX_NB
                for k in range(_HV):
                    acc = buf[r0, pl.ds(k * 16, 16)]
                    for j in range(1, MAX_NB):
                        acc = acc + buf[r0 + j, pl.ds(k * 16, 16)]
                    acc_v[i * _A_PER_CH + a, pl.ds(k * 16, 16)] = acc

        _start(0, buf0, sem0)

        def outer(g, carry):
            i0 = g * 2
            _start(i0 + 1, buf1, sem1)
            _wait(buf0, sem0)
            _reduce(i0, buf0)

            @pl.when(i0 + 2 < n_ch)
            def _():
                _start(i0 + 2, buf0, sem0)

            _wait(buf1, sem1)
            _reduce(i0 + 1, buf1)
            return carry

        lax.fori_loop(0, n_ch // 2, outer, 0)
        pltpu.sync_copy(acc_v, out_hbm.at[pl.ds(wid * a_per_w, a_per_w)])

    return gsum_k(table, idx2d)


def _sc_delta(table, a_msg, idx_rev2d, idx_b2a2d):
    D = table.shape[1]
    rows_per_w = NB_PAD // NW
    n_ch = rows_per_w // CH
    mesh = plsc.VectorSubcoreMesh(core_axis_name="c", subcore_axis_name="s")

    @functools.partial(
        pl.kernel,
        out_type=jax.ShapeDtypeStruct((NB_PAD, D), jnp.float32),
        mesh=mesh,
        compiler_params=pltpu.CompilerParams(use_tc_tiling_on_sc=False),
        scratch_types=[
            pltpu.VMEM((n_ch, CH), jnp.int32),
            pltpu.VMEM((n_ch, CH), jnp.int32),
            pltpu.VMEM((CH, D), jnp.float32),
            pltpu.VMEM((CH, D), jnp.float32),
            pltpu.VMEM((CH, D), jnp.float32),
            pltpu.VMEM((CH, D), jnp.float32),
            pltpu.VMEM((CH, D), jnp.float32),
            pltpu.VMEM_SHARED((NA_PAD, D), jnp.float32),
            pltpu.SemaphoreType.DMA,
            pltpu.SemaphoreType.DMA,
            pltpu.SemaphoreType.DMA,
            pltpu.SemaphoreType.DMA,
        ],
    )
    def delta_k(table_hbm, amsg_hbm, rev_hbm, b2a_hbm, out_hbm,
                irev_v, ib2a_v, rb0, rb1, ab0, ab1, ob,
                shared, sr0, sr1, sa0, sa1):
        wid = lax.axis_index("s") * NC + lax.axis_index("c")
        base = wid * rows_per_w

        @pl.when(lax.axis_index("s") == 0)
        def _():
            pltpu.sync_copy(amsg_hbm, shared)

        pltpu.sync_copy(rev_hbm.at[pl.ds(wid * n_ch, n_ch)], irev_v)
        pltpu.sync_copy(b2a_hbm.at[pl.ds(wid * n_ch, n_ch)], ib2a_v)
        plsc.subcore_barrier()

        def _start(i, rb, ab, sr, sa):
            pltpu.async_copy(table_hbm.at[irev_v.at[i]], rb, sr)
            pltpu.async_copy(shared.at[ib2a_v.at[i]], ab, sa)

        def _wait(rb, ab, sr, sa):
            pltpu.make_async_copy(table_hbm.at[pl.ds(0, CH)], rb, sr).wait()
            pltpu.make_async_copy(table_hbm.at[pl.ds(0, CH)], ab, sa).wait()

        def _emit(i, rb, ab):
            for r in range(CH):
                for k in range(_HV):
                    ob[r, pl.ds(k * 16, 16)] = (
                        ab[r, pl.ds(k * 16, 16)] - rb[r, pl.ds(k * 16, 16)])
            pltpu.sync_copy(ob, out_hbm.at[pl.ds(base + i * CH, CH)])

        _start(0, rb0, ab0, sr0, sa0)

        def outer(g, carry):
            i0 = g * 2
            _start(i0 + 1, rb1, ab1, sr1, sa1)
            _wait(rb0, ab0, sr0, sa0)
            _emit(i0, rb0, ab0)

            @pl.when(i0 + 2 < n_ch)
            def _():
                _start(i0 + 2, rb0, ab0, sr0, sa0)

            _wait(rb1, ab1, sr1, sa1)
            _emit(i0 + 1, rb1, ab1)
            return carry

        lax.fori_loop(0, n_ch // 2, outer, 0)

    return delta_k(table, a_msg, idx_rev2d, idx_b2a2d)


def _mm_in(f_bonds, w1b_t, bg1, wg2_t, bg2):
    blk = 2560
    grid = N_BONDS // blk

    def body(fb_ref, w_ref, b1_ref, w2_ref, b2_ref, fbp_ref, msg_ref):
        fbp = jnp.dot(fb_ref[...], w_ref[...], preferred_element_type=jnp.float32)
        fbp = fbp + b1_ref[...]
        fbp_ref[...] = fbp
        h = jnp.maximum(fbp, 0.0)
        m = jnp.dot(h, w2_ref[...], preferred_element_type=jnp.float32) + b2_ref[...]
        rows = lax.broadcasted_iota(jnp.int32, m.shape, 0)
        m = jnp.where(jnp.logical_and(rows == 0, pl.program_id(0) == 0), 0.0, m)
        msg_ref[...] = m

    return pl.pallas_call(
        body,
        grid=(grid,),
        in_specs=[
            pl.BlockSpec((blk, BOND_FDIM), lambda i: (i, 0)),
            pl.BlockSpec((BOND_FDIM, HIDDEN), lambda i: (0, 0)),
            pl.BlockSpec((1, HIDDEN), lambda i: (0, 0)),
            pl.BlockSpec((HIDDEN, HIDDEN), lambda i: (0, 0)),
            pl.BlockSpec((1, HIDDEN), lambda i: (0, 0)),
        ],
        out_specs=[
            pl.BlockSpec((blk, HIDDEN), lambda i: (i, 0)),
            pl.BlockSpec((blk, HIDDEN), lambda i: (i, 0)),
        ],
        out_shape=[
            jax.ShapeDtypeStruct((NB_PAD, HIDDEN), jnp.float32),
            jax.ShapeDtypeStruct((NB_PAD, HIDDEN), jnp.float32),
        ],
    )(f_bonds, w1b_t, bg1, wg2_t, bg2)


def _depth_update(fbp, delta, wmh_t, wg2_t, bg2):
    grid = NB_PAD // BOND_BLK

    def body(fbp_ref, d_ref, wm_ref, w2_ref, b2_ref, out_ref):
        h = fbp_ref[...] + jnp.dot(d_ref[...], wm_ref[...], preferred_element_type=jnp.float32)
        h = jnp.maximum(h, 0.0)
        m = jnp.dot(h, w2_ref[...], preferred_element_type=jnp.float32) + b2_ref[...]
        rows = lax.broadcasted_iota(jnp.int32, m.shape, 0)
        m = jnp.where(jnp.logical_and(rows == 0, pl.program_id(0) == 0), 0.0, m)
        out_ref[...] = m

    return pl.pallas_call(
        body,
        grid=(grid,),
        in_specs=[
            pl.BlockSpec((BOND_BLK, HIDDEN), lambda i: (i, 0)),
            pl.BlockSpec((BOND_BLK, HIDDEN), lambda i: (i, 0)),
            pl.BlockSpec((HIDDEN, HIDDEN), lambda i: (0, 0)),
            pl.BlockSpec((HIDDEN, HIDDEN), lambda i: (0, 0)),
            pl.BlockSpec((1, HIDDEN), lambda i: (0, 0)),
        ],
        out_specs=pl.BlockSpec((BOND_BLK, HIDDEN), lambda i: (i, 0)),
        out_shape=jax.ShapeDtypeStruct((NB_PAD, HIDDEN), jnp.float32),
    )(fbp, delta, wmh_t, wg2_t, bg2)


def _final_mlp(msgs, wm1_t, bm1, wm2_t, bm2):
    grid = NB_PAD // BOND_BLK
    H2 = 2 * HIDDEN

    def body(m0, m1, m2, m3, w1_ref, b1_ref, w2_ref, b2_ref, out_ref):
        s = jnp.dot(m0[...], w1_ref[0 * HIDDEN:1 * HIDDEN, :], preferred_element_type=jnp.float32)
        s += jnp.dot(m1[...], w1_ref[1 * HIDDEN:2 * HIDDEN, :], preferred_element_type=jnp.float32)
        s += jnp.dot(m2[...], w1_ref[2 * HIDDEN:3 * HIDDEN, :], preferred_element_type=jnp.float32)
        s += jnp.dot(m3[...], w1_ref[3 * HIDDEN:4 * HIDDEN, :], preferred_element_type=jnp.float32)
        h = jnp.maximum(s + b1_ref[...], 0.0)
        out_ref[...] = jnp.dot(h, w2_ref[...], preferred_element_type=jnp.float32) + b2_ref[...]

    mspec = pl.BlockSpec((BOND_BLK, HIDDEN), lambda i: (i, 0))
    return pl.pallas_call(
        body,
        grid=(grid,),
        in_specs=[
            mspec, mspec, mspec, mspec,
            pl.BlockSpec((DEPTH * HIDDEN, H2), lambda i: (0, 0)),
            pl.BlockSpec((1, H2), lambda i: (0, 0)),
            pl.BlockSpec((H2, HIDDEN), lambda i: (0, 0)),
            pl.BlockSpec((1, HIDDEN), lambda i: (0, 0)),
        ],
        out_specs=pl.BlockSpec((BOND_BLK, HIDDEN), lambda i: (i, 0)),
        out_shape=jax.ShapeDtypeStruct((NB_PAD, HIDDEN), jnp.float32),
    )(*msgs, wm1_t, bm1, wm2_t, bm2)


def _out_layer(a_sum, fa_pad, woa_t, wom_t, bo):
    grid = NA_PAD // ATOM_BLK

    def body(g_ref, fa_ref, wa_ref, wm_ref, b_ref, out_ref):
        x = jnp.dot(fa_ref[...], wa_ref[...], preferred_element_type=jnp.float32)
        x += jnp.dot(g_ref[...], wm_ref[...], preferred_element_type=jnp.float32)
        out_ref[...] = jnp.maximum(x + b_ref[...], 0.0)

    return pl.pallas_call(
        body,
        grid=(grid,),
        in_specs=[
            pl.BlockSpec((ATOM_BLK, HIDDEN), lambda i: (i, 0)),
            pl.BlockSpec((ATOM_BLK, ATOM_FDIM), lambda i: (i, 0)),
            pl.BlockSpec((ATOM_FDIM, HIDDEN), lambda i: (0, 0)),
            pl.BlockSpec((HIDDEN, HIDDEN), lambda i: (0, 0)),
            pl.BlockSpec((1, HIDDEN), lambda i: (0, 0)),
        ],
        out_specs=pl.BlockSpec((ATOM_BLK, HIDDEN), lambda i: (i, 0)),
        out_shape=jax.ShapeDtypeStruct((NA_PAD, HIDDEN), jnp.float32),
    )(a_sum, fa_pad, woa_t, wom_t, bo)


def kernel(f_atoms, f_bonds, a2b, b2a, b2revb, undirected_b2a,
           W_g1, b_g1, W_g2, b_g2, W_m1, b_m1, W_m2, b_m2, W_o, b_o):
    del undirected_b2a
    w1b_t = W_g1[:, :BOND_FDIM].T
    wmh_t = W_g1[:, BOND_FDIM:].T
    wg2_t = W_g2.T
    wm1_t = W_m1.T
    wm2_t = W_m2.T
    woa_t = W_o[:, :ATOM_FDIM].T
    wom_t = W_o[:, ATOM_FDIM:].T
    bg1 = b_g1[None, :]
    bg2 = b_g2[None, :]
    bm1 = b_m1[None, :]
    bm2 = b_m2[None, :]
    bo = b_o[None, :]

    a2b_p = jnp.pad(a2b, ((0, NA_PAD - N_ATOMS), (0, 0)))
    a2b2d = a2b_p.reshape(-1, CH)
    rev2d = jnp.pad(b2revb, (0, NB_PAD - N_BONDS)).reshape(-1, CH)
    b2a2d = jnp.pad(b2a, (0, NB_PAD - N_BONDS)).reshape(-1, CH)
    fa_pad = jnp.pad(f_atoms, ((0, NA_PAD - N_ATOMS), (0, 0)))

    fbp, msg = _mm_in(f_bonds, w1b_t, bg1, wg2_t, bg2)
    msgs = [msg]
    for _ in range(DEPTH - 1):
        a_msg = _sc_gather_sum(msg, a2b2d)
        delta = _sc_delta(msg, a_msg, rev2d, b2a2d)
        msg = _depth_update(fbp, delta, wmh_t, wg2_t, bg2)
        msgs.append(msg)

    tmp = _final_mlp(msgs, wm1_t, bm1, wm2_t, bm2)
    a_sum = _sc_gather_sum(tmp, a2b2d)
    out_pad = _out_layer(a_sum, fa_pad, woa_t, wom_t, bo)
    return out_pad[:N_ATOMS]
